# baseline (speedup 1.0000x reference)
import functools

import jax
import jax.numpy as jnp
from jax import lax
from jax.experimental import pallas as pl
from jax.experimental.pallas import tpu as pltpu
from jax.experimental.pallas import tpu_sc as plsc


@functools.cache
def _make_lookup(V, D, B1, B2):
    info = plsc.get_sparse_core_info()
    NC = info.num_cores
    L = info.num_lanes
    NW = NC * info.num_subcores
    planes_per_w = D // NW
    assert D % NW == 0 and B1 % L == 0
    n_grp = B1 // L

    mesh = plsc.VectorSubcoreMesh(core_axis_name="c", subcore_axis_name="s")

    @functools.partial(
        pl.kernel,
        mesh=mesh,
        compiler_params=pltpu.CompilerParams(
            use_tc_tiling_on_sc=True, needs_layout_passes=False
        ),
        out_type=jax.ShapeDtypeStruct((B2, D, B1), jnp.float32),
        scratch_types=[
            pltpu.VMEM((V,), jnp.float32),
            pltpu.VMEM((B1,), jnp.int32),
            pltpu.VMEM((B1,), jnp.int32),
            pltpu.VMEM((B1,), jnp.int32),
            pltpu.VMEM((B1,), jnp.float32),
            pltpu.VMEM((B1,), jnp.float32),
            pltpu.VMEM((B1,), jnp.float32),
            pltpu.SemaphoreType.DMA,
            pltpu.SemaphoreType.DMA,
            pltpu.SemaphoreType.DMA,
            pltpu.SemaphoreType.DMA,
            pltpu.SemaphoreType.DMA,
            pltpu.SemaphoreType.DMA,
        ],
    )
    def lookup(tok_hbm, wt_hbm, out_hbm, plane_v, tok0_v, tok1_v, tok2_v,
               row0_v, row1_v, row2_v, tsem0, tsem1, tsem2,
               osem0, osem1, osem2):
        toks = (tok0_v, tok1_v, tok2_v)
        rows = (row0_v, row1_v, row2_v)
        tsems = (tsem0, tsem1, tsem2)
        osems = (osem0, osem1, osem2)
        R = 3
        wid = lax.axis_index("s") * NC + lax.axis_index("c")

        UNROLL = 32

        def gather_row(tok_b, row_b):
            def grp(g, carry):
                for u in range(UNROLL):
                    sl = pl.ds((g * UNROLL + u) * L, L)
                    row_b[sl] = plsc.load_gather(plane_v, [tok_b[sl]])
                return carry
            lax.fori_loop(0, n_grp // UNROLL, grp, 0)

        for dd in range(planes_per_w):
            d = wid * planes_per_w + dd
            pltpu.sync_copy(wt_hbm.at[d], plane_v)
            for b in range(R):
                pltpu.async_copy(tok_hbm.at[b], toks[b], tsems[b])

            def body(s, b):
                pltpu.make_async_copy(
                    tok_hbm.at[0], toks[b], tsems[b]
                ).wait()

                @pl.when(s >= R)
                def _():
                    pltpu.make_async_copy(
                        rows[b], out_hbm.at[0, 0], osems[b]
                    ).wait()

                gather_row(toks[b], rows[b])
                sn = s + R

                @pl.when(sn < B2)
                def _():
                    pltpu.async_copy(tok_hbm.at[sn], toks[b], tsems[b])

                pltpu.async_copy(rows[b], out_hbm.at[s, d], osems[b])

            def step(g, carry):
                for k in range(R):
                    body(g * R + k, k)
                return carry

            lax.fori_loop(0, B2 // R, step, 0)
            for s in range(B2 - B2 % R, B2):
                body(jnp.int32(s), s % R)
            for b in range(R):
                pltpu.make_async_copy(
                    rows[b], out_hbm.at[0, 0], osems[b]
                ).wait()

    return lookup


def kernel(token_ids, weight):
    B1, B2 = token_ids.shape
    V, D = weight.shape
    out_t = _make_lookup(V, D, B1, B2)(
        token_ids.astype(jnp.int32).T, weight.T
    )
    return out_t.transpose(2, 0, 1)


# speedup vs baseline: 1.0134x; 1.0134x over previous
import functools

import jax
import jax.numpy as jnp
from jax import lax
from jax.experimental import pallas as pl
from jax.experimental.pallas import tpu as pltpu
from jax.experimental.pallas import tpu_sc as plsc


@functools.cache
def _make_lookup(V, D, B1, B2):
    info = plsc.get_sparse_core_info()
    NC = info.num_cores
    L = info.num_lanes
    NW = NC * info.num_subcores
    planes_per_w = D // NW
    assert D % NW == 0 and B1 % L == 0
    n_grp = B1 // L

    mesh = plsc.VectorSubcoreMesh(core_axis_name="c", subcore_axis_name="s")

    @functools.partial(
        pl.kernel,
        mesh=mesh,
        compiler_params=pltpu.CompilerParams(
            use_tc_tiling_on_sc=True, needs_layout_passes=False
        ),
        out_type=jax.ShapeDtypeStruct((B2, D, B1), jnp.float32),
        scratch_types=[
            pltpu.VMEM((V,), jnp.float32),
            pltpu.VMEM((B1,), jnp.int32),
            pltpu.VMEM((B1,), jnp.int32),
            pltpu.VMEM((B1,), jnp.int32),
            pltpu.VMEM((B1,), jnp.float32),
            pltpu.VMEM((B1,), jnp.float32),
            pltpu.VMEM((B1,), jnp.float32),
            pltpu.SemaphoreType.DMA,
            pltpu.SemaphoreType.DMA,
            pltpu.SemaphoreType.DMA,
            pltpu.SemaphoreType.DMA,
            pltpu.SemaphoreType.DMA,
            pltpu.SemaphoreType.DMA,
        ],
    )
    def lookup(tok_hbm, wt_hbm, out_hbm, plane_v, tok0_v, tok1_v, tok2_v,
               row0_v, row1_v, row2_v, tsem0, tsem1, tsem2,
               osem0, osem1, osem2):
        toks = (tok0_v, tok1_v, tok2_v)
        rows = (row0_v, row1_v, row2_v)
        tsems = (tsem0, tsem1, tsem2)
        osems = (osem0, osem1, osem2)
        R = 3
        wid = lax.axis_index("s") * NC + lax.axis_index("c")

        UNROLL = 16

        def gather_row(tok_b, row_b):
            def grp(g, carry):
                for u in range(UNROLL):
                    sl = pl.ds((g * UNROLL + u) * L, L)
                    row_b[sl] = plsc.load_gather(plane_v, [tok_b[sl]])
                return carry
            lax.fori_loop(0, n_grp // UNROLL, grp, 0)

        for dd in range(planes_per_w):
            d = wid * planes_per_w + dd
            pltpu.sync_copy(wt_hbm.at[d], plane_v)
            for b in range(R):
                pltpu.async_copy(tok_hbm.at[b], toks[b], tsems[b])

            def body(s, b):
                pltpu.make_async_copy(
                    tok_hbm.at[0], toks[b], tsems[b]
                ).wait()

                @pl.when(s >= R)
                def _():
                    pltpu.make_async_copy(
                        rows[b], out_hbm.at[0, 0], osems[b]
                    ).wait()

                gather_row(toks[b], rows[b])
                sn = s + R

                @pl.when(sn < B2)
                def _():
                    pltpu.async_copy(tok_hbm.at[sn], toks[b], tsems[b])

                pltpu.async_copy(rows[b], out_hbm.at[s, d], osems[b])

            def step(g, carry):
                for k in range(R):
                    body(g * R + k, k)
                return carry

            lax.fori_loop(0, B2 // R, step, 0)
            for s in range(B2 - B2 % R, B2):
                body(jnp.int32(s), s % R)
            for b in range(R):
                pltpu.make_async_copy(
                    rows[b], out_hbm.at[0, 0], osems[b]
                ).wait()

    return lookup


def kernel(token_ids, weight):
    B1, B2 = token_ids.shape
    V, D = weight.shape
    out_t = _make_lookup(V, D, B1, B2)(
        token_ids.astype(jnp.int32).T, weight.T
    )
    return out_t.transpose(2, 0, 1)
